# Initial kernel scaffold; baseline (speedup 1.0000x reference)
import functools

import jax
import jax.numpy as jnp
from jax import lax
from jax.experimental import pallas as pl
from jax.experimental.pallas import tpu as pltpu
from jax.experimental.pallas import tpu_sc as plsc


def _pre_body(p1_ref, w0_ref, b0_ref, w1_ref, b1_ref, h_ref):
    t = jnp.dot(p1_ref[...], w0_ref[...], preferred_element_type=jnp.float32)
    t = jnp.tanh(t + b0_ref[...])
    t = jnp.dot(t, w1_ref[...], preferred_element_type=jnp.float32)
    h_ref[...] = jnp.tanh(t + b1_ref[...])


def _pre_ff(p1, w0, b0, w1, b1):
    n, d = p1.shape
    return pl.pallas_call(
        _pre_body,
        out_shape=jax.ShapeDtypeStruct((n, d), jnp.float32),
    )(p1, w0, b0.reshape(1, d), w1, b1.reshape(1, d))


def _make_gather(n_rows, d):
    info = plsc.get_sparse_core_info()
    nc, ns = info.num_cores, info.num_subcores
    nw = nc * ns
    per_w = n_rows // nw
    sub = 80
    grp = 10
    blk = sub * grp
    ngrp = per_w // blk
    assert per_w % blk == 0

    def body(h_hbm, idx_hbm, out_hbm, idxb, rows, sem):
        c = lax.axis_index("c")
        s = lax.axis_index("s")
        wid = s * nc + c
        base = wid * per_w

        def group(g, carry):
            off = base + g * blk
            pltpu.sync_copy(idx_hbm.at[pl.ds(off, blk)], idxb)
            cps = [
                pltpu.async_copy(
                    h_hbm.at[idxb.at[pl.ds(k * sub, sub)]],
                    rows.at[pl.ds(k * sub, sub)],
                    sem,
                )
                for k in range(grp)
            ]
            for cp in cps:
                cp.wait()
            pltpu.sync_copy(rows, out_hbm.at[pl.ds(off, blk)])
            return carry

        lax.fori_loop(0, ngrp, group, 0)

    mesh = plsc.VectorSubcoreMesh(core_axis_name="c", subcore_axis_name="s")
    return pl.kernel(
        body,
        out_type=jax.ShapeDtypeStruct((n_rows, d), jnp.float32),
        mesh=mesh,
        scratch_types=[
            pltpu.VMEM((blk,), jnp.int32),
            pltpu.VMEM((blk, d), jnp.float32),
            pltpu.SemaphoreType.DMA,
        ],
    )


def _pair_body(nb, d, hi_ref, hj_ref, basis_ref, w2_ref, b2_ref,
               ii0_ref, ii1_ref, eqw_ref, eqb_ref, i1_ref, i1g_ref):
    cat = jnp.concatenate([hi_ref[...], hj_ref[...]], axis=1)
    inter = jnp.dot(cat, w2_ref[...], preferred_element_type=jnp.float32)
    inter = jnp.tanh(inter + b2_ref[...])
    acc = inter[:, 0:d] * basis_ref[:, 0:1]
    for b in range(1, nb):
        acc = acc + inter[:, b * d:(b + 1) * d] * basis_ref[:, b:b + 1]
    i1 = jnp.tanh(jnp.dot(acc, ii0_ref[...], preferred_element_type=jnp.float32))
    i1 = jnp.tanh(jnp.dot(i1, ii1_ref[...], preferred_element_type=jnp.float32))
    i1_ref[...] = i1
    g = jnp.dot(i1, eqw_ref[...], preferred_element_type=jnp.float32)
    i1g_ref[...] = jnp.tanh(g + eqb_ref[...])


def _pair_ff(hc, basis, w2, b2, ii0, ii1, eqw, eqb, n_pairs, d, nb, bp):
    nblk = n_pairs // bp
    full = lambda *shape: pl.BlockSpec(shape, lambda m: (0,) * len(shape))
    return pl.pallas_call(
        functools.partial(_pair_body, nb, d),
        grid=(nblk,),
        in_specs=[
            pl.BlockSpec((bp, d), lambda m: (m, 0)),
            pl.BlockSpec((bp, d), lambda m: (m + nblk, 0)),
            pl.BlockSpec((bp, nb), lambda m: (m, 0)),
            full(2 * d, d * nb),
            full(1, d * nb),
            full(d, d),
            full(d, d),
            full(d, d),
            full(1, d),
        ],
        out_specs=[
            pl.BlockSpec((bp, d), lambda m: (m, 0)),
            pl.BlockSpec((bp, d), lambda m: (m, 0)),
        ],
        out_shape=[
            jax.ShapeDtypeStruct((n_pairs, d), jnp.float32),
            jax.ShapeDtypeStruct((n_pairs, d), jnp.float32),
        ],
    )(hc, hc, basis, w2, b2, ii0, ii1, eqw, eqb)


def _make_scatter(n_atoms, n_pairs, d):
    info = plsc.get_sparse_core_info()
    nc, ns = info.num_cores, info.num_subcores
    half = n_pairs // nc
    per_tile = half // ns
    sub = 80
    nsub = 5
    blk = sub * nsub
    nblk = per_tile // blk
    assert per_tile % blk == 0
    slab_rows = n_atoms // ns
    zr = 125
    nz = slab_rows // zr
    assert slab_rows % zr == 0
    nv = d // 16

    def body(i1_hbm, i1g_hbm, p30, p31, p32, d30, d31, d32,
             i2d_hbm, j2d_hbm, s_hbm,
             acc, ib2, jb2, db, rows, g1, zb, sem):
        c = lax.axis_index("c")
        s = lax.axis_index("s")
        pbase = c * half + s * per_tile
        rowbase = pbase // sub
        slab = s * slab_rows

        def zero_zb():
            def zrow(r, carry):
                for v in range(nv):
                    zb[r, pl.ds(v * 16, 16)] = jnp.zeros((16,), jnp.float32)
                return carry
            lax.fori_loop(0, zr, zrow, 0)

        def zero_acc():
            for z in range(nz):
                pltpu.sync_copy(zb, acc.at[pl.ds(slab + z * zr, zr)])

        def dump(phase):
            pltpu.sync_copy(acc.at[pl.ds(slab, slab_rows)],
                            s_hbm.at[phase, c, pl.ds(slab, slab_rows)])

        zero_zb()
        zero_acc()
        plsc.subcore_barrier()

        def inv_blk(bk, carry):
            off = pbase + bk * blk
            pltpu.sync_copy(i2d_hbm.at[pl.ds(rowbase + bk * nsub, nsub)], ib2)
            pltpu.sync_copy(i1_hbm.at[pl.ds(off, blk)], rows)
            for r in range(nsub):
                pltpu.sync_copy(rows.at[pl.ds(r * sub, sub)],
                                acc.at[ib2.at[r]], add=True)
            return carry

        lax.fori_loop(0, nblk, inv_blk, 0)
        plsc.subcore_barrier()
        dump(0)
        plsc.subcore_barrier()

        for x, (px, dx) in enumerate(((p30, d30), (p31, d31), (p32, d32))):
            zero_acc()
            plsc.subcore_barrier()

            def eq_blk(bk, carry, px=px, dx=dx):
                off = pbase + bk * blk
                rb = rowbase + bk * nsub
                pltpu.sync_copy(i2d_hbm.at[pl.ds(rb, nsub)], ib2)
                pltpu.sync_copy(j2d_hbm.at[pl.ds(rb, nsub)], jb2)
                cps = [
                    pltpu.async_copy(px.at[jb2.at[r]],
                                     rows.at[pl.ds(r * sub, sub)], sem)
                    for r in range(nsub)
                ]
                pltpu.sync_copy(i1g_hbm.at[pl.ds(off, blk)], g1)
                pltpu.sync_copy(dx.at[pl.ds(off, blk)], db)
                for cp in cps:
                    cp.wait()

                def pair(k, carry2):
                    dvec = jnp.full((16,), db[k], jnp.float32)
                    for v in range(nv):
                        sl = pl.ds(v * 16, 16)
                        rows[k, sl] = (rows[k, sl] + dvec) * g1[k, sl]
                    return carry2

                lax.fori_loop(0, blk, pair, 0)
                for r in range(nsub):
                    pltpu.sync_copy(rows.at[pl.ds(r * sub, sub)],
                                    acc.at[ib2.at[r]], add=True)
                return carry

            lax.fori_loop(0, nblk, eq_blk, 0)
            plsc.subcore_barrier()
            dump(1 + x)
            plsc.subcore_barrier()

    mesh = plsc.VectorSubcoreMesh(core_axis_name="c", subcore_axis_name="s")
    return pl.kernel(
        body,
        out_type=jax.ShapeDtypeStruct((4, nc, n_atoms, d), jnp.float32),
        mesh=mesh,
        scratch_types=[
            pltpu.VMEM_SHARED((n_atoms, d), jnp.float32),
            pltpu.VMEM((nsub, sub), jnp.int32),
            pltpu.VMEM((nsub, sub), jnp.int32),
            pltpu.VMEM((blk,), jnp.float32),
            pltpu.VMEM((blk, d), jnp.float32),
            pltpu.VMEM((blk, d), jnp.float32),
            pltpu.VMEM((zr, d), jnp.float32),
            pltpu.SemaphoreType.DMA,
        ],
    )


def _fin_body(s_ref, pw0_ref, pw1_ref, ew0_ref, ew1_ref, o1_ref, o3_ref):
    t = s_ref[0, 0] + s_ref[0, 1]
    t = jnp.dot(t, pw0_ref[...], preferred_element_type=jnp.float32)
    o1_ref[...] = jnp.dot(t, pw1_ref[...], preferred_element_type=jnp.float32)
    for x in range(3):
        u = s_ref[1 + x, 0] + s_ref[1 + x, 1]
        u = jnp.dot(u, ew0_ref[...], preferred_element_type=jnp.float32)
        u = jnp.dot(u, ew1_ref[...], preferred_element_type=jnp.float32)
        o3_ref[:, x, :] = u


def _finalize(s, pw0, pw1, ew0, ew1, n_atoms, d, ab):
    full = lambda *shape: pl.BlockSpec(shape, lambda m: (0,) * len(shape))
    return pl.pallas_call(
        _fin_body,
        grid=(n_atoms // ab,),
        in_specs=[
            pl.BlockSpec((4, 2, ab, d), lambda m: (0, 0, m, 0)),
            full(d, d), full(d, d), full(d, d), full(d, d),
        ],
        out_specs=[
            pl.BlockSpec((ab, d), lambda m: (m, 0)),
            pl.BlockSpec((ab, 3, d), lambda m: (m, 0, 0)),
        ],
        out_shape=[
            jax.ShapeDtypeStruct((n_atoms, d), jnp.float32),
            jax.ShapeDtypeStruct((n_atoms, 3, d), jnp.float32),
        ],
    )(s, pw0, pw1, ew0, ew1)


def kernel(p1, p3, d3, basis, ind_2, pp_pre_W0, pp_pre_b0, pp_pre_W1,
           pp_pre_b1, pi_W0, pi_b0, ii_W0, ii_W1, pp_post_W0, pp_post_W1,
           eq_ii_W0, eq_ii_b0, eq_pp_W0, eq_pp_W1):
    n_atoms, d = p1.shape
    n_pairs = ind_2.shape[0]
    nb = basis.shape[1]

    i_idx = ind_2[:, 0]
    j_idx = ind_2[:, 1]

    h = _pre_ff(p1, pp_pre_W0, pp_pre_b0, pp_pre_W1, pp_pre_b1)

    idx_all = jnp.concatenate([i_idx, j_idx])
    hc = _make_gather(2 * n_pairs, d)(h, idx_all)

    w2 = pi_W0.reshape(2 * d, d, nb).transpose(0, 2, 1).reshape(2 * d, d * nb)
    b2 = pi_b0.reshape(d, nb).T.reshape(1, d * nb)
    i1, i1g = _pair_ff(hc, basis, w2, b2, ii_W0, ii_W1, eq_ii_W0,
                       eq_ii_b0.reshape(1, d), n_pairs, d, nb, bp=640)

    sub = 80
    i2d = i_idx.reshape(n_pairs // sub, sub)
    j2d = j_idx.reshape(n_pairs // sub, sub)
    s = _make_scatter(n_atoms, n_pairs, d)(
        i1, i1g,
        p3[:, 0, :], p3[:, 1, :], p3[:, 2, :],
        d3[:, 0], d3[:, 1], d3[:, 2],
        i2d, j2d,
    )

    p1_new, p3_new = _finalize(s, pp_post_W0, pp_post_W1, eq_pp_W0,
                               eq_pp_W1, n_atoms, d, ab=2000)
    return (p1_new, p3_new)


# speedup vs baseline: 13.1574x; 13.1574x over previous
import functools

import jax
import jax.numpy as jnp
from jax import lax
from jax.experimental import pallas as pl
from jax.experimental.pallas import tpu as pltpu
from jax.experimental.pallas import tpu_sc as plsc


def _pre_body(p1_ref, w0_ref, b0_ref, w1_ref, b1_ref, h_ref):
    t = jnp.dot(p1_ref[...], w0_ref[...], preferred_element_type=jnp.float32)
    t = jnp.tanh(t + b0_ref[...])
    t = jnp.dot(t, w1_ref[...], preferred_element_type=jnp.float32)
    h_ref[...] = jnp.tanh(t + b1_ref[...])


def _pre_ff(p1, w0, b0, w1, b1):
    n, d = p1.shape
    return pl.pallas_call(
        _pre_body,
        out_shape=jax.ShapeDtypeStruct((n, d), jnp.float32),
    )(p1, w0, b0.reshape(1, d), w1, b1.reshape(1, d))


def _make_gather(n_rows, d):
    info = plsc.get_sparse_core_info()
    nc, ns = info.num_cores, info.num_subcores
    nw = nc * ns
    per_w = n_rows // nw
    sub = 80
    grp = 10
    blk = sub * grp
    ngrp = per_w // blk
    assert per_w % blk == 0

    def body(h_hbm, idx_hbm, out_hbm, idxb, rows, sem):
        c = lax.axis_index("c")
        s = lax.axis_index("s")
        wid = s * nc + c
        base = wid * per_w

        def group(g, carry):
            off = pl.multiple_of(base + g * blk, 8)
            pltpu.sync_copy(idx_hbm.at[pl.ds(off, blk)], idxb)
            cps = [
                pltpu.async_copy(
                    h_hbm.at[idxb.at[pl.ds(k * sub, sub)]],
                    rows.at[pl.ds(k * sub, sub)],
                    sem,
                )
                for k in range(grp)
            ]
            for cp in cps:
                cp.wait()
            pltpu.sync_copy(rows, out_hbm.at[pl.ds(off, blk)])
            return carry

        lax.fori_loop(0, ngrp, group, 0)

    mesh = plsc.VectorSubcoreMesh(core_axis_name="c", subcore_axis_name="s")
    return pl.kernel(
        body,
        out_type=jax.ShapeDtypeStruct((n_rows, d), jnp.float32),
        mesh=mesh,
        scratch_types=[
            pltpu.VMEM((blk,), jnp.int32),
            pltpu.VMEM((blk, d), jnp.float32),
            pltpu.SemaphoreType.DMA,
        ],
    )


def _pair_body(nb, d, hi_ref, hj_ref, basis_ref, w2_ref, b2_ref,
               ii0_ref, ii1_ref, eqw_ref, eqb_ref, i1_ref, i1g_ref):
    cat = jnp.concatenate([hi_ref[...], hj_ref[...]], axis=1)
    inter = jnp.dot(cat, w2_ref[...], preferred_element_type=jnp.float32)
    inter = jnp.tanh(inter + b2_ref[...])
    acc = inter[:, 0:d] * basis_ref[:, 0:1]
    for b in range(1, nb):
        acc = acc + inter[:, b * d:(b + 1) * d] * basis_ref[:, b:b + 1]
    i1 = jnp.tanh(jnp.dot(acc, ii0_ref[...], preferred_element_type=jnp.float32))
    i1 = jnp.tanh(jnp.dot(i1, ii1_ref[...], preferred_element_type=jnp.float32))
    i1_ref[...] = i1
    g = jnp.dot(i1, eqw_ref[...], preferred_element_type=jnp.float32)
    i1g_ref[...] = jnp.tanh(g + eqb_ref[...])


def _pair_ff(hc, basis, w2, b2, ii0, ii1, eqw, eqb, n_pairs, d, nb, bp):
    nblk = n_pairs // bp
    full = lambda *shape: pl.BlockSpec(shape, lambda m: (0,) * len(shape))
    return pl.pallas_call(
        functools.partial(_pair_body, nb, d),
        grid=(nblk,),
        in_specs=[
            pl.BlockSpec((bp, d), lambda m: (m, 0)),
            pl.BlockSpec((bp, d), lambda m: (m + nblk, 0)),
            pl.BlockSpec((bp, nb), lambda m: (m, 0)),
            full(2 * d, d * nb),
            full(1, d * nb),
            full(d, d),
            full(d, d),
            full(d, d),
            full(1, d),
        ],
        out_specs=[
            pl.BlockSpec((bp, d), lambda m: (m, 0)),
            pl.BlockSpec((bp, d), lambda m: (m, 0)),
        ],
        out_shape=[
            jax.ShapeDtypeStruct((n_pairs, d), jnp.float32),
            jax.ShapeDtypeStruct((n_pairs, d), jnp.float32),
        ],
    )(hc, hc, basis, w2, b2, ii0, ii1, eqw, eqb)


def _make_scatter(n_atoms, n_pairs, d):
    info = plsc.get_sparse_core_info()
    nc, ns = info.num_cores, info.num_subcores
    nw = nc * ns
    sub = 80
    nsub = 8
    sbk = sub * nsub
    nsbk_total = n_pairs // sbk
    assert n_pairs % sbk == 0
    base_cnt = nsbk_total // nw
    extra = nsbk_total - base_cnt * nw
    zr = 128
    slab_rows = ((n_atoms + ns - 1) // ns + zr - 1) // zr * zr
    nacc = ns * slab_rows
    nz = slab_rows // zr
    nv = d // 16

    def body(i1_hbm, i1g_hbm, p30, p31, p32, d30, d31, d32,
             i2d_hbm, j2d_hbm, s_hbm,
             acc, ib2, jb2, db, rows, g1, zb, sem):
        c = lax.axis_index("c")
        s = lax.axis_index("s")
        w = s * nc + c
        nsbk = base_cnt + jnp.where(w < extra, 1, 0)
        slab = s * slab_rows

        def zero_zb():
            def zrow(r, carry):
                for v in range(nv):
                    zb[r, pl.ds(v * 16, 16)] = jnp.zeros((16,), jnp.float32)
                return carry
            lax.fori_loop(0, zr, zrow, 0)

        def zero_acc():
            for z in range(nz):
                pltpu.sync_copy(zb, acc.at[pl.ds(slab + z * zr, zr)])

        def dump(phase):
            pltpu.sync_copy(acc.at[pl.ds(slab, slab_rows)],
                            s_hbm.at[phase, c, pl.ds(slab, slab_rows)])

        zero_zb()
        zero_acc()
        plsc.subcore_barrier()

        def inv_sbk(k, carry):
            sb = w + nw * k
            rb = pl.multiple_of(sb * nsub, 8)
            pltpu.sync_copy(i2d_hbm.at[pl.ds(rb, nsub)], ib2)
            for r in range(nsub):
                off = pl.multiple_of(sb * sbk + r * sub, 8)
                pltpu.sync_copy(i1_hbm.at[pl.ds(off, sub)], rows)
                pltpu.sync_copy(rows, acc.at[ib2.at[r]], add=True)
            return carry

        lax.fori_loop(0, nsbk, inv_sbk, 0)
        plsc.subcore_barrier()
        dump(0)
        plsc.subcore_barrier()

        for x, (px, dx) in enumerate(((p30, d30), (p31, d31), (p32, d32))):
            zero_acc()
            plsc.subcore_barrier()

            def eq_sbk(k, carry, px=px, dx=dx):
                sb = w + nw * k
                rb = pl.multiple_of(sb * nsub, 8)
                pltpu.sync_copy(i2d_hbm.at[pl.ds(rb, nsub)], ib2)
                pltpu.sync_copy(j2d_hbm.at[pl.ds(rb, nsub)], jb2)
                for r in range(nsub):
                    off = pl.multiple_of(sb * sbk + r * sub, 8)
                    gcp = pltpu.async_copy(px.at[jb2.at[r]], rows, sem)
                    pltpu.sync_copy(i1g_hbm.at[pl.ds(off, sub)], g1)
                    pltpu.sync_copy(dx.at[pl.ds(off, sub)], db.at[pl.ds(0, sub)])
                    gcp.wait()

                    def pair(k2, carry2):
                        dvec = jnp.full((16,), db[pl.ds(k2, 16)][0],
                                        jnp.float32)
                        for v in range(nv):
                            sl = pl.ds(v * 16, 16)
                            rows[k2, sl] = (rows[k2, sl] + dvec) * g1[k2, sl]
                        return carry2

                    lax.fori_loop(0, sub, pair, 0)
                    pltpu.sync_copy(rows, acc.at[ib2.at[r]], add=True)
                return carry

            lax.fori_loop(0, nsbk, eq_sbk, 0)
            plsc.subcore_barrier()
            dump(1 + x)
            plsc.subcore_barrier()

    mesh = plsc.VectorSubcoreMesh(core_axis_name="c", subcore_axis_name="s")
    return pl.kernel(
        body,
        out_type=jax.ShapeDtypeStruct((4, nc, nacc, d), jnp.float32),
        mesh=mesh,
        scratch_types=[
            pltpu.VMEM_SHARED((nacc, d), jnp.float32),
            pltpu.VMEM((nsub, sub), jnp.int32),
            pltpu.VMEM((nsub, sub), jnp.int32),
            pltpu.VMEM((sub + 16,), jnp.float32),
            pltpu.VMEM((sub, d), jnp.float32),
            pltpu.VMEM((sub, d), jnp.float32),
            pltpu.VMEM((zr, d), jnp.float32),
            pltpu.SemaphoreType.DMA,
        ],
    )


def _fin_body(s_ref, pw0_ref, pw1_ref, ew0_ref, ew1_ref, o1_ref, o3_ref):
    t = s_ref[0, 0] + s_ref[0, 1]
    t = jnp.dot(t, pw0_ref[...], preferred_element_type=jnp.float32)
    o1_ref[...] = jnp.dot(t, pw1_ref[...], preferred_element_type=jnp.float32)
    for x in range(3):
        u = s_ref[1 + x, 0] + s_ref[1 + x, 1]
        u = jnp.dot(u, ew0_ref[...], preferred_element_type=jnp.float32)
        u = jnp.dot(u, ew1_ref[...], preferred_element_type=jnp.float32)
        o3_ref[:, x, :] = u


def _finalize(s, pw0, pw1, ew0, ew1, n_atoms, d, ab):
    full = lambda *shape: pl.BlockSpec(shape, lambda m: (0,) * len(shape))
    return pl.pallas_call(
        _fin_body,
        grid=(n_atoms // ab,),
        in_specs=[
            pl.BlockSpec((4, 2, ab, d), lambda m: (0, 0, m, 0)),
            full(d, d), full(d, d), full(d, d), full(d, d),
        ],
        out_specs=[
            pl.BlockSpec((ab, d), lambda m: (m, 0)),
            pl.BlockSpec((ab, 3, d), lambda m: (m, 0, 0)),
        ],
        out_shape=[
            jax.ShapeDtypeStruct((n_atoms, d), jnp.float32),
            jax.ShapeDtypeStruct((n_atoms, 3, d), jnp.float32),
        ],
    )(s, pw0, pw1, ew0, ew1)


def kernel(p1, p3, d3, basis, ind_2, pp_pre_W0, pp_pre_b0, pp_pre_W1,
           pp_pre_b1, pi_W0, pi_b0, ii_W0, ii_W1, pp_post_W0, pp_post_W1,
           eq_ii_W0, eq_ii_b0, eq_pp_W0, eq_pp_W1):
    n_atoms, d = p1.shape
    n_pairs = ind_2.shape[0]
    nb = basis.shape[1]

    i_idx = ind_2[:, 0]
    j_idx = ind_2[:, 1]

    h = _pre_ff(p1, pp_pre_W0, pp_pre_b0, pp_pre_W1, pp_pre_b1)

    idx_all = jnp.concatenate([i_idx, j_idx])
    hc = _make_gather(2 * n_pairs, d)(h, idx_all)

    w2 = pi_W0.reshape(2 * d, d, nb).transpose(0, 2, 1).reshape(2 * d, d * nb)
    b2 = pi_b0.reshape(d, nb).T.reshape(1, d * nb)
    i1, i1g = _pair_ff(hc, basis, w2, b2, ii_W0, ii_W1, eq_ii_W0,
                       eq_ii_b0.reshape(1, d), n_pairs, d, nb, bp=640)

    sub = 80
    i2d = i_idx.reshape(n_pairs // sub, sub)
    j2d = j_idx.reshape(n_pairs // sub, sub)
    s = _make_scatter(n_atoms, n_pairs, d)(
        i1, i1g,
        p3[:, 0, :], p3[:, 1, :], p3[:, 2, :],
        d3[:, 0], d3[:, 1], d3[:, 2],
        i2d, j2d,
    )

    p1_new, p3_new = _finalize(s, pp_post_W0, pp_post_W1, eq_pp_W0,
                               eq_pp_W1, n_atoms, d, ab=2000)
    return (p1_new, p3_new)


# speedup vs baseline: 13.2359x; 1.0060x over previous
import functools

import jax
import jax.numpy as jnp
from jax import lax
from jax.experimental import pallas as pl
from jax.experimental.pallas import tpu as pltpu
from jax.experimental.pallas import tpu_sc as plsc


def _pre_body(p1_ref, w0_ref, b0_ref, w1_ref, b1_ref, h_ref):
    t = jnp.dot(p1_ref[...], w0_ref[...], preferred_element_type=jnp.float32)
    t = jnp.tanh(t + b0_ref[...])
    t = jnp.dot(t, w1_ref[...], preferred_element_type=jnp.float32)
    h_ref[...] = jnp.tanh(t + b1_ref[...])


def _pre_ff(p1, w0, b0, w1, b1):
    n, d = p1.shape
    return pl.pallas_call(
        _pre_body,
        out_shape=jax.ShapeDtypeStruct((n, d), jnp.float32),
    )(p1, w0, b0.reshape(1, d), w1, b1.reshape(1, d))


def _make_gather(n_rows, d):
    info = plsc.get_sparse_core_info()
    nc, ns = info.num_cores, info.num_subcores
    nw = nc * ns
    per_w = n_rows // nw
    sub = 80
    grp = 10
    blk = sub * grp
    ngrp = per_w // blk
    assert per_w % blk == 0

    def body(h_hbm, idx_hbm, out_hbm, idxb, rows, sem):
        c = lax.axis_index("c")
        s = lax.axis_index("s")
        wid = s * nc + c
        base = wid * per_w

        def group(g, carry):
            off = pl.multiple_of(base + g * blk, 8)
            pltpu.sync_copy(idx_hbm.at[pl.ds(off, blk)], idxb)
            cps = [
                pltpu.async_copy(
                    h_hbm.at[idxb.at[pl.ds(k * sub, sub)]],
                    rows.at[pl.ds(k * sub, sub)],
                    sem,
                )
                for k in range(grp)
            ]
            for cp in cps:
                cp.wait()
            pltpu.sync_copy(rows, out_hbm.at[pl.ds(off, blk)])
            return carry

        lax.fori_loop(0, ngrp, group, 0)

    mesh = plsc.VectorSubcoreMesh(core_axis_name="c", subcore_axis_name="s")
    return pl.kernel(
        body,
        out_type=jax.ShapeDtypeStruct((n_rows, d), jnp.float32),
        mesh=mesh,
        scratch_types=[
            pltpu.VMEM((blk,), jnp.int32),
            pltpu.VMEM((blk, d), jnp.float32),
            pltpu.SemaphoreType.DMA,
        ],
    )


def _pair_body(nb, d, hi_ref, hj_ref, basis_ref, w2_ref, b2_ref,
               ii0_ref, ii1_ref, eqw_ref, eqb_ref, i1_ref, i1g_ref):
    cat = jnp.concatenate([hi_ref[...], hj_ref[...]], axis=1)
    inter = jnp.dot(cat.astype(jnp.bfloat16), w2_ref[...],
                    preferred_element_type=jnp.float32)
    inter = jnp.tanh(inter + b2_ref[...])
    acc = inter[:, 0:d] * basis_ref[:, 0:1]
    for b in range(1, nb):
        acc = acc + inter[:, b * d:(b + 1) * d] * basis_ref[:, b:b + 1]
    i1 = jnp.tanh(jnp.dot(acc, ii0_ref[...], preferred_element_type=jnp.float32))
    i1 = jnp.tanh(jnp.dot(i1, ii1_ref[...], preferred_element_type=jnp.float32))
    i1_ref[...] = i1
    g = jnp.dot(i1, eqw_ref[...], preferred_element_type=jnp.float32)
    i1g_ref[...] = jnp.tanh(g + eqb_ref[...])


def _pair_ff(hc, basis, w2, b2, ii0, ii1, eqw, eqb, n_pairs, d, nb, bp):
    nblk = n_pairs // bp
    full = lambda *shape: pl.BlockSpec(shape, lambda m: (0,) * len(shape))
    return pl.pallas_call(
        functools.partial(_pair_body, nb, d),
        grid=(nblk,),
        in_specs=[
            pl.BlockSpec((bp, d), lambda m: (m, 0)),
            pl.BlockSpec((bp, d), lambda m: (m + nblk, 0)),
            pl.BlockSpec((bp, nb), lambda m: (m, 0)),
            full(2 * d, d * nb),
            full(1, d * nb),
            full(d, d),
            full(d, d),
            full(d, d),
            full(1, d),
        ],
        out_specs=[
            pl.BlockSpec((bp, d), lambda m: (m, 0)),
            pl.BlockSpec((bp, d), lambda m: (m, 0)),
        ],
        out_shape=[
            jax.ShapeDtypeStruct((n_pairs, d), jnp.float32),
            jax.ShapeDtypeStruct((n_pairs, d), jnp.float32),
        ],
    )(hc, hc, basis, w2, b2, ii0, ii1, eqw, eqb)


def _make_scatter(n_atoms, n_pairs, d):
    info = plsc.get_sparse_core_info()
    nc, ns = info.num_cores, info.num_subcores
    nw = nc * ns
    sub = 80
    nsub = 8
    sbk = sub * nsub
    nsbk_total = n_pairs // sbk
    assert n_pairs % sbk == 0
    base_cnt = nsbk_total // nw
    extra = nsbk_total - base_cnt * nw
    zr = 128
    slab_rows = ((n_atoms + ns - 1) // ns + zr - 1) // zr * zr
    nacc = ns * slab_rows
    nz = slab_rows // zr
    nv = d // 16

    def body(i1_hbm, i1g_hbm, p30, p31, p32, d30, d31, d32,
             i2d_hbm, j2d_hbm, s_hbm,
             acc, ib2, jb2, db, rows, g1, zb, sem):
        c = lax.axis_index("c")
        s = lax.axis_index("s")
        w = s * nc + c
        nsbk = base_cnt + jnp.where(w < extra, 1, 0)
        slab = s * slab_rows

        def zero_zb():
            def zrow(r, carry):
                for v in range(nv):
                    zb[r, pl.ds(v * 16, 16)] = jnp.zeros((16,), jnp.float32)
                return carry
            lax.fori_loop(0, zr, zrow, 0)

        def zero_acc():
            for z in range(nz):
                pltpu.sync_copy(zb, acc.at[pl.ds(slab + z * zr, zr)])

        def dump(phase):
            pltpu.sync_copy(acc.at[pl.ds(slab, slab_rows)],
                            s_hbm.at[phase, c, pl.ds(slab, slab_rows)])

        zero_zb()
        zero_acc()
        plsc.subcore_barrier()

        def inv_sbk(k, carry):
            sb = w + nw * k
            rb = pl.multiple_of(sb * nsub, 8)
            pltpu.sync_copy(i2d_hbm.at[pl.ds(rb, nsub)], ib2)
            for r in range(nsub):
                off = pl.multiple_of(sb * sbk + r * sub, 8)
                pltpu.sync_copy(i1_hbm.at[pl.ds(off, sub)], rows)
                pltpu.sync_copy(rows, acc.at[ib2.at[r]], add=True)
            return carry

        lax.fori_loop(0, nsbk, inv_sbk, 0)
        plsc.subcore_barrier()
        dump(0)
        plsc.subcore_barrier()

        for x, (px, dx) in enumerate(((p30, d30), (p31, d31), (p32, d32))):
            zero_acc()
            plsc.subcore_barrier()

            def eq_sbk(k, carry, px=px, dx=dx):
                sb = w + nw * k
                rb = pl.multiple_of(sb * nsub, 8)
                pltpu.sync_copy(i2d_hbm.at[pl.ds(rb, nsub)], ib2)
                pltpu.sync_copy(j2d_hbm.at[pl.ds(rb, nsub)], jb2)
                for r in range(nsub):
                    off = pl.multiple_of(sb * sbk + r * sub, 8)
                    gcp = pltpu.async_copy(px.at[jb2.at[r]], rows, sem)
                    pltpu.sync_copy(i1g_hbm.at[pl.ds(off, sub)], g1)
                    pltpu.sync_copy(dx.at[pl.ds(off, sub)], db.at[pl.ds(0, sub)])
                    gcp.wait()

                    def pair(k2, carry2):
                        dvec = jnp.full((16,), db[pl.ds(k2, 16)][0],
                                        jnp.float32)
                        for v in range(nv):
                            sl = pl.ds(v * 16, 16)
                            rows[k2, sl] = (rows[k2, sl] + dvec) * g1[k2, sl]
                        return carry2

                    lax.fori_loop(0, sub, pair, 0)
                    pltpu.sync_copy(rows, acc.at[ib2.at[r]], add=True)
                return carry

            lax.fori_loop(0, nsbk, eq_sbk, 0)
            plsc.subcore_barrier()
            dump(1 + x)
            plsc.subcore_barrier()

    mesh = plsc.VectorSubcoreMesh(core_axis_name="c", subcore_axis_name="s")
    return pl.kernel(
        body,
        out_type=jax.ShapeDtypeStruct((4, nc, nacc, d), jnp.float32),
        mesh=mesh,
        scratch_types=[
            pltpu.VMEM_SHARED((nacc, d), jnp.float32),
            pltpu.VMEM((nsub, sub), jnp.int32),
            pltpu.VMEM((nsub, sub), jnp.int32),
            pltpu.VMEM((sub + 16,), jnp.float32),
            pltpu.VMEM((sub, d), jnp.float32),
            pltpu.VMEM((sub, d), jnp.float32),
            pltpu.VMEM((zr, d), jnp.float32),
            pltpu.SemaphoreType.DMA,
        ],
    )


def _fin_body(s_ref, pw0_ref, pw1_ref, ew0_ref, ew1_ref, o1_ref, o3_ref):
    t = s_ref[0, 0] + s_ref[0, 1]
    t = jnp.dot(t, pw0_ref[...], preferred_element_type=jnp.float32)
    o1_ref[...] = jnp.dot(t, pw1_ref[...], preferred_element_type=jnp.float32)
    for x in range(3):
        u = s_ref[1 + x, 0] + s_ref[1 + x, 1]
        u = jnp.dot(u, ew0_ref[...], preferred_element_type=jnp.float32)
        u = jnp.dot(u, ew1_ref[...], preferred_element_type=jnp.float32)
        o3_ref[:, x, :] = u


def _finalize(s, pw0, pw1, ew0, ew1, n_atoms, d, ab):
    full = lambda *shape: pl.BlockSpec(shape, lambda m: (0,) * len(shape))
    return pl.pallas_call(
        _fin_body,
        grid=(n_atoms // ab,),
        in_specs=[
            pl.BlockSpec((4, 2, ab, d), lambda m: (0, 0, m, 0)),
            full(d, d), full(d, d), full(d, d), full(d, d),
        ],
        out_specs=[
            pl.BlockSpec((ab, d), lambda m: (m, 0)),
            pl.BlockSpec((ab, 3, d), lambda m: (m, 0, 0)),
        ],
        out_shape=[
            jax.ShapeDtypeStruct((n_atoms, d), jnp.float32),
            jax.ShapeDtypeStruct((n_atoms, 3, d), jnp.float32),
        ],
    )(s, pw0, pw1, ew0, ew1)


def kernel(p1, p3, d3, basis, ind_2, pp_pre_W0, pp_pre_b0, pp_pre_W1,
           pp_pre_b1, pi_W0, pi_b0, ii_W0, ii_W1, pp_post_W0, pp_post_W1,
           eq_ii_W0, eq_ii_b0, eq_pp_W0, eq_pp_W1):
    n_atoms, d = p1.shape
    n_pairs = ind_2.shape[0]
    nb = basis.shape[1]

    i_idx = ind_2[:, 0]
    j_idx = ind_2[:, 1]

    h = _pre_ff(p1, pp_pre_W0, pp_pre_b0, pp_pre_W1, pp_pre_b1)

    idx_all = jnp.concatenate([i_idx, j_idx])
    hc = _make_gather(2 * n_pairs, d)(h, idx_all)

    w2 = pi_W0.reshape(2 * d, d, nb).transpose(0, 2, 1).reshape(2 * d, d * nb)
    w2 = w2.astype(jnp.bfloat16)
    b2 = pi_b0.reshape(d, nb).T.reshape(1, d * nb)
    i1, i1g = _pair_ff(hc, basis, w2, b2, ii_W0, ii_W1, eq_ii_W0,
                       eq_ii_b0.reshape(1, d), n_pairs, d, nb, bp=640)

    sub = 80
    i2d = i_idx.reshape(n_pairs // sub, sub)
    j2d = j_idx.reshape(n_pairs // sub, sub)
    s = _make_scatter(n_atoms, n_pairs, d)(
        i1, i1g,
        p3[:, 0, :], p3[:, 1, :], p3[:, 2, :],
        d3[:, 0], d3[:, 1], d3[:, 2],
        i2d, j2d,
    )

    p1_new, p3_new = _finalize(s, pp_post_W0, pp_post_W1, eq_pp_W0,
                               eq_pp_W1, n_atoms, d, ab=2000)
    return (p1_new, p3_new)
